# baseline (speedup 1.0000x reference)
import functools

import jax
import jax.numpy as jnp
from jax import lax
from jax.experimental import pallas as pl
from jax.experimental.pallas import tpu as pltpu
from jax.experimental.pallas import tpu_sc as plsc

B = 4
C = 96
H = 512
W = 512
CT = 56
C_SC = C - CT
NW = 32
HW_ = H // NW
PW = HW_ * W
NV = PW // 16
CB = 4
NJ = C_SC // (2 * CB)
HB = 64

OFF_A = 0
OFF_S = C_SC * 16
OFF_Q = 2 * C_SC * 16
OFF_NOC = 3 * C_SC * 16
OFF_NK = OFF_NOC + 16
ACC_LEN = OFF_NK + 16


def _sc_body(feat_hbm, odoc_hbm, kink_hbm, out_hbm,
             modoc, mkink, woc, wkb,
             fb00, fb01, fb02, fb03, fb10, fb11, fb12, fb13,
             accb, sem0, sem1):
    fb0 = (fb00, fb01, fb02, fb03)
    fb1 = (fb10, fb11, fb12, fb13)
    wid = lax.axis_index("s") * 2 + lax.axis_index("c")
    h0 = wid * HW_
    z16 = jnp.zeros((16,), jnp.float32)

    def zero_body(j, _):
        accb[pl.ds(j * 16, 16)] = z16
        return 0

    lax.fori_loop(0, ACC_LEN // 16, zero_body, 0)

    def compute(bufs, c0):
        def pix_body(i, carry):
            accs = list(carry)
            r = i >> 5
            o = (i & 31) * 16
            wo = woc[r, pl.ds(o, 16)]
            wk = wkb[r, pl.ds(o, 16)]
            for k in range(CB):
                f = bufs[k][r, pl.ds(o, 16)]
                t = f * wk
                accs[3 * k + 0] = accs[3 * k + 0] + f * wo
                accs[3 * k + 1] = accs[3 * k + 1] + t
                accs[3 * k + 2] = accs[3 * k + 2] + t * t
            return tuple(accs)

        res = lax.fori_loop(0, NV, pix_body, (z16,) * (3 * CB), unroll=4)
        for k in range(CB):
            o = (c0 + k) * 16
            accb[pl.ds(OFF_A + o, 16)] = accb[pl.ds(OFF_A + o, 16)] \
                + res[3 * k + 0]
            accb[pl.ds(OFF_S + o, 16)] = accb[pl.ds(OFF_S + o, 16)] \
                + res[3 * k + 1]
            accb[pl.ds(OFF_Q + o, 16)] = accb[pl.ds(OFF_Q + o, 16)] \
                + res[3 * k + 2]

    for b in range(B):
        pltpu.sync_copy(odoc_hbm.at[b, pl.ds(h0, HW_), :], modoc)
        pltpu.sync_copy(kink_hbm.at[b, pl.ds(h0, HW_), :], mkink)

        def conv_body(i, carry):
            c_oc, c_k = carry
            r = i >> 5
            o = (i & 31) * 16
            mo = modoc[r, pl.ds(o, 16)]
            mk = mkink[r, pl.ds(o, 16)]
            wo = jnp.where(mo == 2, 1.0, 0.0).astype(jnp.float32)
            wki = jnp.where(mk == 1, 1.0, 0.0).astype(jnp.float32)
            woc[r, pl.ds(o, 16)] = wo
            wkb[r, pl.ds(o, 16)] = wki
            return c_oc + wo, c_k + wki

        c_oc, c_k = lax.fori_loop(0, NV, conv_body, (z16, z16), unroll=2)
        accb[pl.ds(OFF_NOC, 16)] = accb[pl.ds(OFF_NOC, 16)] + c_oc
        accb[pl.ds(OFF_NK, 16)] = accb[pl.ds(OFF_NK, 16)] + c_k

        for k in range(CB):
            pltpu.async_copy(
                feat_hbm.at[b, CT + k, pl.ds(h0, HW_), :], fb0[k], sem0)

        def j_body(j, _):
            c0 = j * 2 * CB
            for k in range(CB):
                pltpu.make_async_copy(
                    feat_hbm.at[b, CT + c0 + k, pl.ds(h0, HW_), :], fb0[k],
                    sem0).wait()
            for k in range(CB):
                pltpu.async_copy(
                    feat_hbm.at[b, CT + c0 + CB + k, pl.ds(h0, HW_), :],
                    fb1[k], sem1)
            compute(fb0, c0)
            for k in range(CB):
                pltpu.make_async_copy(
                    feat_hbm.at[b, CT + c0 + CB + k, pl.ds(h0, HW_), :],
                    fb1[k], sem1).wait()

            @pl.when(j < NJ - 1)
            def _():
                for k in range(CB):
                    pltpu.async_copy(
                        feat_hbm.at[b, CT + c0 + 2 * CB + k,
                                    pl.ds(h0, HW_), :],
                        fb0[k], sem0)

            compute(fb1, c0 + CB)
            return 0

        lax.fori_loop(0, NJ, j_body, 0)

    pltpu.sync_copy(accb, out_hbm.at[wid])


def _tc_body(feat_ref, odoc_ref, kink_ref, out_ref):
    b = pl.program_id(0)
    hstep = pl.program_id(1)

    @pl.when(jnp.logical_and(b == 0, hstep == 0))
    def _():
        out_ref[...] = jnp.zeros_like(out_ref)

    f = feat_ref[0]
    wo = (odoc_ref[0] == 2).astype(jnp.float32)
    wk = (kink_ref[0] == 1).astype(jnp.float32)
    t = f * wk[None]
    a = jnp.sum(f * wo[None], axis=(1, 2))
    s = jnp.sum(t, axis=(1, 2))
    q = jnp.sum(t * t, axis=(1, 2))
    out_ref[...] = out_ref[...] + jnp.stack([a, s, q])


@jax.jit
def kernel(features, odoc_mask, kink_mask):
    mesh = plsc.VectorSubcoreMesh(core_axis_name="c", subcore_axis_name="s")
    sc_partials = pl.kernel(
        _sc_body,
        mesh=mesh,
        out_type=jax.ShapeDtypeStruct((NW, ACC_LEN), jnp.float32),
        scratch_types=[
            pltpu.VMEM((HW_, W), jnp.int32),
            pltpu.VMEM((HW_, W), jnp.int32),
            pltpu.VMEM((HW_, W), jnp.float32),
            pltpu.VMEM((HW_, W), jnp.float32),
        ] + [pltpu.VMEM((HW_, W), jnp.float32) for _ in range(2 * CB)]
        + [
            pltpu.VMEM((ACC_LEN,), jnp.float32),
            pltpu.SemaphoreType.DMA,
            pltpu.SemaphoreType.DMA,
        ],
    )(features, odoc_mask, kink_mask)

    tc_partials = pl.pallas_call(
        _tc_body,
        grid=(B, H // HB),
        in_specs=[
            pl.BlockSpec((1, CT, HB, W), lambda b, h: (b, 0, h, 0)),
            pl.BlockSpec((1, HB, W), lambda b, h: (b, h, 0)),
            pl.BlockSpec((1, HB, W), lambda b, h: (b, h, 0)),
        ],
        out_specs=pl.BlockSpec((3, CT), lambda b, h: (0, 0)),
        out_shape=jax.ShapeDtypeStruct((3, CT), jnp.float32),
    )(features, odoc_mask, kink_mask)

    tot = jnp.sum(sc_partials, axis=0)
    a_sc = jnp.sum(tot[OFF_A:OFF_S].reshape(C_SC, 16), axis=1)
    s_sc = jnp.sum(tot[OFF_S:OFF_Q].reshape(C_SC, 16), axis=1)
    q_sc = jnp.sum(tot[OFF_Q:OFF_NOC].reshape(C_SC, 16), axis=1)
    n_oc = jnp.sum(tot[OFF_NOC:OFF_NK])
    n_kink = jnp.sum(tot[OFF_NK:ACC_LEN])

    a = jnp.concatenate([tc_partials[0], a_sc])
    s = jnp.concatenate([tc_partials[1], s_sc])
    q_sum = jnp.sum(tc_partials[2]) + jnp.sum(q_sc)

    center = a / n_oc
    num = n_kink * jnp.sum(center * center) - 2.0 * jnp.sum(center * s) \
        + q_sum
    mse = num / (n_kink * C)
    return jnp.where(jnp.isnan(mse), jnp.zeros((), jnp.float32), mse)


# speedup vs baseline: 1.0199x; 1.0199x over previous
import functools

import jax
import jax.numpy as jnp
from jax import lax
from jax.experimental import pallas as pl
from jax.experimental.pallas import tpu as pltpu
from jax.experimental.pallas import tpu_sc as plsc

B = 4
C = 96
H = 512
W = 512
CT = 56
C_SC = C - CT
NW = 32
HW_ = H // NW
PW = HW_ * W
NV = PW // 16
CB = 4
NJ = C_SC // (2 * CB)
HB = 32

OFF_A = 0
OFF_S = C_SC * 16
OFF_Q = 2 * C_SC * 16
OFF_NOC = 3 * C_SC * 16
OFF_NK = OFF_NOC + 16
ACC_LEN = OFF_NK + 16


def _sc_body(feat_hbm, odoc_hbm, kink_hbm, out_hbm,
             modoc, mkink, woc, wkb,
             fb00, fb01, fb02, fb03, fb10, fb11, fb12, fb13,
             accb, sem0, sem1):
    fb0 = (fb00, fb01, fb02, fb03)
    fb1 = (fb10, fb11, fb12, fb13)
    wid = lax.axis_index("s") * 2 + lax.axis_index("c")
    h0 = wid * HW_
    z16 = jnp.zeros((16,), jnp.float32)

    def zero_body(j, _):
        accb[pl.ds(j * 16, 16)] = z16
        return 0

    lax.fori_loop(0, ACC_LEN // 16, zero_body, 0)

    def compute(bufs, c0):
        def pix_body(i, carry):
            accs = list(carry)
            r = i >> 5
            o = (i & 31) * 16
            wo = woc[r, pl.ds(o, 16)]
            wk = wkb[r, pl.ds(o, 16)]
            for k in range(CB):
                f = bufs[k][r, pl.ds(o, 16)]
                t = f * wk
                accs[3 * k + 0] = accs[3 * k + 0] + f * wo
                accs[3 * k + 1] = accs[3 * k + 1] + t
                accs[3 * k + 2] = accs[3 * k + 2] + t * t
            return tuple(accs)

        res = lax.fori_loop(0, NV, pix_body, (z16,) * (3 * CB), unroll=4)
        for k in range(CB):
            o = (c0 + k) * 16
            accb[pl.ds(OFF_A + o, 16)] = accb[pl.ds(OFF_A + o, 16)] \
                + res[3 * k + 0]
            accb[pl.ds(OFF_S + o, 16)] = accb[pl.ds(OFF_S + o, 16)] \
                + res[3 * k + 1]
            accb[pl.ds(OFF_Q + o, 16)] = accb[pl.ds(OFF_Q + o, 16)] \
                + res[3 * k + 2]

    for b in range(B):
        pltpu.sync_copy(odoc_hbm.at[b, pl.ds(h0, HW_), :], modoc)
        pltpu.sync_copy(kink_hbm.at[b, pl.ds(h0, HW_), :], mkink)

        def conv_body(i, carry):
            c_oc, c_k = carry
            r = i >> 5
            o = (i & 31) * 16
            mo = modoc[r, pl.ds(o, 16)]
            mk = mkink[r, pl.ds(o, 16)]
            wo = jnp.where(mo == 2, 1.0, 0.0).astype(jnp.float32)
            wki = jnp.where(mk == 1, 1.0, 0.0).astype(jnp.float32)
            woc[r, pl.ds(o, 16)] = wo
            wkb[r, pl.ds(o, 16)] = wki
            return c_oc + wo, c_k + wki

        c_oc, c_k = lax.fori_loop(0, NV, conv_body, (z16, z16), unroll=2)
        accb[pl.ds(OFF_NOC, 16)] = accb[pl.ds(OFF_NOC, 16)] + c_oc
        accb[pl.ds(OFF_NK, 16)] = accb[pl.ds(OFF_NK, 16)] + c_k

        for k in range(CB):
            pltpu.async_copy(
                feat_hbm.at[b, CT + k, pl.ds(h0, HW_), :], fb0[k], sem0)

        def j_body(j, _):
            c0 = j * 2 * CB
            for k in range(CB):
                pltpu.make_async_copy(
                    feat_hbm.at[b, CT + c0 + k, pl.ds(h0, HW_), :], fb0[k],
                    sem0).wait()
            for k in range(CB):
                pltpu.async_copy(
                    feat_hbm.at[b, CT + c0 + CB + k, pl.ds(h0, HW_), :],
                    fb1[k], sem1)
            compute(fb0, c0)
            for k in range(CB):
                pltpu.make_async_copy(
                    feat_hbm.at[b, CT + c0 + CB + k, pl.ds(h0, HW_), :],
                    fb1[k], sem1).wait()

            @pl.when(j < NJ - 1)
            def _():
                for k in range(CB):
                    pltpu.async_copy(
                        feat_hbm.at[b, CT + c0 + 2 * CB + k,
                                    pl.ds(h0, HW_), :],
                        fb0[k], sem0)

            compute(fb1, c0 + CB)
            return 0

        lax.fori_loop(0, NJ, j_body, 0)

    pltpu.sync_copy(accb, out_hbm.at[wid])


def _tc_body(feat_ref, odoc_ref, kink_ref, out_ref):
    b = pl.program_id(0)
    hstep = pl.program_id(1)

    @pl.when(jnp.logical_and(b == 0, hstep == 0))
    def _():
        out_ref[...] = jnp.zeros_like(out_ref)

    f = feat_ref[0]
    wo = (odoc_ref[0] == 2).astype(jnp.float32)
    wk = (kink_ref[0] == 1).astype(jnp.float32)
    t = f * wk[None]
    a = jnp.sum(f * wo[None], axis=(1, 2))
    s = jnp.sum(t, axis=(1, 2))
    q = jnp.sum(t * t, axis=(1, 2))
    out_ref[...] = out_ref[...] + jnp.stack([a, s, q])


@jax.jit
def kernel(features, odoc_mask, kink_mask):
    mesh = plsc.VectorSubcoreMesh(core_axis_name="c", subcore_axis_name="s")
    sc_partials = pl.kernel(
        _sc_body,
        mesh=mesh,
        out_type=jax.ShapeDtypeStruct((NW, ACC_LEN), jnp.float32),
        scratch_types=[
            pltpu.VMEM((HW_, W), jnp.int32),
            pltpu.VMEM((HW_, W), jnp.int32),
            pltpu.VMEM((HW_, W), jnp.float32),
            pltpu.VMEM((HW_, W), jnp.float32),
        ] + [pltpu.VMEM((HW_, W), jnp.float32) for _ in range(2 * CB)]
        + [
            pltpu.VMEM((ACC_LEN,), jnp.float32),
            pltpu.SemaphoreType.DMA,
            pltpu.SemaphoreType.DMA,
        ],
    )(features, odoc_mask, kink_mask)

    tc_partials = pl.pallas_call(
        _tc_body,
        grid=(B, H // HB),
        in_specs=[
            pl.BlockSpec((1, CT, HB, W), lambda b, h: (b, 0, h, 0)),
            pl.BlockSpec((1, HB, W), lambda b, h: (b, h, 0)),
            pl.BlockSpec((1, HB, W), lambda b, h: (b, h, 0)),
        ],
        out_specs=pl.BlockSpec((3, CT), lambda b, h: (0, 0)),
        out_shape=jax.ShapeDtypeStruct((3, CT), jnp.float32),
    )(features, odoc_mask, kink_mask)

    tot = jnp.sum(sc_partials, axis=0)
    a_sc = jnp.sum(tot[OFF_A:OFF_S].reshape(C_SC, 16), axis=1)
    s_sc = jnp.sum(tot[OFF_S:OFF_Q].reshape(C_SC, 16), axis=1)
    q_sc = jnp.sum(tot[OFF_Q:OFF_NOC].reshape(C_SC, 16), axis=1)
    n_oc = jnp.sum(tot[OFF_NOC:OFF_NK])
    n_kink = jnp.sum(tot[OFF_NK:ACC_LEN])

    a = jnp.concatenate([tc_partials[0], a_sc])
    s = jnp.concatenate([tc_partials[1], s_sc])
    q_sum = jnp.sum(tc_partials[2]) + jnp.sum(q_sc)

    center = a / n_oc
    num = n_kink * jnp.sum(center * center) - 2.0 * jnp.sum(center * s) \
        + q_sum
    mse = num / (n_kink * C)
    return jnp.where(jnp.isnan(mse), jnp.zeros((), jnp.float32), mse)


# speedup vs baseline: 1.0307x; 1.0106x over previous
import functools

import jax
import jax.numpy as jnp
from jax import lax
from jax.experimental import pallas as pl
from jax.experimental.pallas import tpu as pltpu
from jax.experimental.pallas import tpu_sc as plsc

B = 4
C = 96
H = 512
W = 512
CT = 56
C_SC = C - CT
NW = 32
HW_ = H // NW
PW = HW_ * W
NV = PW // 16
CB = 4
NJ = C_SC // (2 * CB)
HB = 32

OFF_A = 0
OFF_S = C_SC * 16
OFF_Q = 2 * C_SC * 16
OFF_NOC = 3 * C_SC * 16
OFF_NK = OFF_NOC + 16
ACC_LEN = OFF_NK + 16


def _sc_body(feat_hbm, odoc_hbm, kink_hbm, out_hbm,
             modoc0, mkink0, modoc1, mkink1, woc, wkb,
             fb00, fb01, fb02, fb03, fb10, fb11, fb12, fb13,
             accb, sem0, sem1, semm):
    fb0 = (fb00, fb01, fb02, fb03)
    fb1 = (fb10, fb11, fb12, fb13)
    mbufs = ((modoc0, mkink0), (modoc1, mkink1))
    wid = lax.axis_index("s") * 2 + lax.axis_index("c")
    h0 = wid * HW_
    z16 = jnp.zeros((16,), jnp.float32)

    def zero_body(j, _):
        accb[pl.ds(j * 16, 16)] = z16
        return 0

    lax.fori_loop(0, ACC_LEN // 16, zero_body, 0)

    def compute(bufs, c0):
        def pix_body(i, carry):
            accs = list(carry)
            r = i >> 5
            o = (i & 31) * 16
            wo = woc[r, pl.ds(o, 16)]
            wk = wkb[r, pl.ds(o, 16)]
            for k in range(CB):
                f = bufs[k][r, pl.ds(o, 16)]
                t = f * wk
                accs[3 * k + 0] = accs[3 * k + 0] + f * wo
                accs[3 * k + 1] = accs[3 * k + 1] + t
                accs[3 * k + 2] = accs[3 * k + 2] + t * t
            return tuple(accs)

        res = lax.fori_loop(0, NV, pix_body, (z16,) * (3 * CB), unroll=4)
        for k in range(CB):
            o = (c0 + k) * 16
            accb[pl.ds(OFF_A + o, 16)] = accb[pl.ds(OFF_A + o, 16)] \
                + res[3 * k + 0]
            accb[pl.ds(OFF_S + o, 16)] = accb[pl.ds(OFF_S + o, 16)] \
                + res[3 * k + 1]
            accb[pl.ds(OFF_Q + o, 16)] = accb[pl.ds(OFF_Q + o, 16)] \
                + res[3 * k + 2]

    pltpu.sync_copy(odoc_hbm.at[0, pl.ds(h0, HW_), :], mbufs[0][0])
    pltpu.sync_copy(kink_hbm.at[0, pl.ds(h0, HW_), :], mbufs[0][1])
    for k in range(CB):
        pltpu.async_copy(
            feat_hbm.at[0, CT + k, pl.ds(h0, HW_), :], fb0[k], sem0)
    pltpu.async_copy(odoc_hbm.at[1, pl.ds(h0, HW_), :], mbufs[1][0], semm)
    pltpu.async_copy(kink_hbm.at[1, pl.ds(h0, HW_), :], mbufs[1][1], semm)

    for b in range(B):
        modoc, mkink = mbufs[b % 2]
        if b > 0:
            pltpu.make_async_copy(
                odoc_hbm.at[b, pl.ds(h0, HW_), :], modoc, semm).wait()
            pltpu.make_async_copy(
                kink_hbm.at[b, pl.ds(h0, HW_), :], mkink, semm).wait()

        def conv_body(i, carry):
            c_oc, c_k = carry
            r = i >> 5
            o = (i & 31) * 16
            mo = modoc[r, pl.ds(o, 16)]
            mk = mkink[r, pl.ds(o, 16)]
            wo = jnp.where(mo == 2, 1.0, 0.0).astype(jnp.float32)
            wki = jnp.where(mk == 1, 1.0, 0.0).astype(jnp.float32)
            woc[r, pl.ds(o, 16)] = wo
            wkb[r, pl.ds(o, 16)] = wki
            return c_oc + wo, c_k + wki

        c_oc, c_k = lax.fori_loop(0, NV, conv_body, (z16, z16), unroll=2)
        accb[pl.ds(OFF_NOC, 16)] = accb[pl.ds(OFF_NOC, 16)] + c_oc
        accb[pl.ds(OFF_NK, 16)] = accb[pl.ds(OFF_NK, 16)] + c_k

        if b + 2 < B:
            pltpu.async_copy(
                odoc_hbm.at[b + 2, pl.ds(h0, HW_), :], mbufs[b % 2][0],
                semm)
            pltpu.async_copy(
                kink_hbm.at[b + 2, pl.ds(h0, HW_), :], mbufs[b % 2][1],
                semm)

        def j_body(j, _):
            c0 = j * 2 * CB
            for k in range(CB):
                pltpu.make_async_copy(
                    feat_hbm.at[b, CT + c0 + k, pl.ds(h0, HW_), :], fb0[k],
                    sem0).wait()
            for k in range(CB):
                pltpu.async_copy(
                    feat_hbm.at[b, CT + c0 + CB + k, pl.ds(h0, HW_), :],
                    fb1[k], sem1)
            compute(fb0, c0)
            for k in range(CB):
                pltpu.make_async_copy(
                    feat_hbm.at[b, CT + c0 + CB + k, pl.ds(h0, HW_), :],
                    fb1[k], sem1).wait()

            @pl.when(j < NJ - 1)
            def _():
                for k in range(CB):
                    pltpu.async_copy(
                        feat_hbm.at[b, CT + c0 + 2 * CB + k,
                                    pl.ds(h0, HW_), :],
                        fb0[k], sem0)

            if b + 1 < B:
                @pl.when(j == NJ - 1)
                def _():
                    for k in range(CB):
                        pltpu.async_copy(
                            feat_hbm.at[b + 1, CT + k, pl.ds(h0, HW_), :],
                            fb0[k], sem0)

            compute(fb1, c0 + CB)
            return 0

        lax.fori_loop(0, NJ, j_body, 0)

    pltpu.sync_copy(accb, out_hbm.at[wid])


def _tc_body(feat_ref, odoc_ref, kink_ref, out_ref):
    b = pl.program_id(0)
    hstep = pl.program_id(1)

    @pl.when(jnp.logical_and(b == 0, hstep == 0))
    def _():
        out_ref[...] = jnp.zeros_like(out_ref)

    f = feat_ref[0]
    wo = (odoc_ref[0] == 2).astype(jnp.float32)
    wk = (kink_ref[0] == 1).astype(jnp.float32)
    t = f * wk[None]
    a = jnp.sum(f * wo[None], axis=(1, 2))
    s = jnp.sum(t, axis=(1, 2))
    q = jnp.sum(t * t, axis=(1, 2))
    out_ref[...] = out_ref[...] + jnp.stack([a, s, q])


@jax.jit
def kernel(features, odoc_mask, kink_mask):
    mesh = plsc.VectorSubcoreMesh(core_axis_name="c", subcore_axis_name="s")
    sc_partials = pl.kernel(
        _sc_body,
        mesh=mesh,
        out_type=jax.ShapeDtypeStruct((NW, ACC_LEN), jnp.float32),
        scratch_types=[
            pltpu.VMEM((HW_, W), jnp.int32),
            pltpu.VMEM((HW_, W), jnp.int32),
            pltpu.VMEM((HW_, W), jnp.int32),
            pltpu.VMEM((HW_, W), jnp.int32),
            pltpu.VMEM((HW_, W), jnp.float32),
            pltpu.VMEM((HW_, W), jnp.float32),
        ] + [pltpu.VMEM((HW_, W), jnp.float32) for _ in range(2 * CB)]
        + [
            pltpu.VMEM((ACC_LEN,), jnp.float32),
            pltpu.SemaphoreType.DMA,
            pltpu.SemaphoreType.DMA,
            pltpu.SemaphoreType.DMA,
        ],
    )(features, odoc_mask, kink_mask)

    tc_partials = pl.pallas_call(
        _tc_body,
        grid=(B, H // HB),
        in_specs=[
            pl.BlockSpec((1, CT, HB, W), lambda b, h: (b, 0, h, 0)),
            pl.BlockSpec((1, HB, W), lambda b, h: (b, h, 0)),
            pl.BlockSpec((1, HB, W), lambda b, h: (b, h, 0)),
        ],
        out_specs=pl.BlockSpec((3, CT), lambda b, h: (0, 0)),
        out_shape=jax.ShapeDtypeStruct((3, CT), jnp.float32),
    )(features, odoc_mask, kink_mask)

    tot = jnp.sum(sc_partials, axis=0)
    a_sc = jnp.sum(tot[OFF_A:OFF_S].reshape(C_SC, 16), axis=1)
    s_sc = jnp.sum(tot[OFF_S:OFF_Q].reshape(C_SC, 16), axis=1)
    q_sc = jnp.sum(tot[OFF_Q:OFF_NOC].reshape(C_SC, 16), axis=1)
    n_oc = jnp.sum(tot[OFF_NOC:OFF_NK])
    n_kink = jnp.sum(tot[OFF_NK:ACC_LEN])

    a = jnp.concatenate([tc_partials[0], a_sc])
    s = jnp.concatenate([tc_partials[1], s_sc])
    q_sum = jnp.sum(tc_partials[2]) + jnp.sum(q_sc)

    center = a / n_oc
    num = n_kink * jnp.sum(center * center) - 2.0 * jnp.sum(center * s) \
        + q_sum
    mse = num / (n_kink * C)
    return jnp.where(jnp.isnan(mse), jnp.zeros((), jnp.float32), mse)
